# Initial kernel scaffold; baseline (speedup 1.0000x reference)
import jax
import jax.numpy as jnp
from jax.experimental import pallas as pl


def kernel(frames):
    raise NotImplementedError("write your pallas kernel here")


# speedup vs baseline: 56.4205x; 56.4205x over previous
import functools

import jax
import jax.numpy as jnp
import numpy as np
from jax.experimental import pallas as pl
from jax.experimental.pallas import tpu as pltpu

_N, _T, _C, _H, _W = 8, 16, 3, 224, 224
_B = _N * _T
_CX = int(_H * 0.5 + 0.5)
_CY = int(_W * 0.5 + 0.5)


def _rects() -> np.ndarray:
    key = jax.random.key(42)
    kx, ky = jax.random.split(key)
    ox = np.asarray(jax.random.randint(kx, (_B,), 0, _H + (1 - _CX % 2)))
    oy = np.asarray(jax.random.randint(ky, (_B,), 0, _W + (1 - _CY % 2)))
    r0 = np.clip(ox - _CX // 2, 0, _H - 1)
    r1 = np.clip(ox - _CX // 2 + _CX - 1, 0, _H - 1)
    c0 = np.clip(oy - _CY // 2, 0, _W - 1)
    c1 = np.clip(oy - _CY // 2 + _CY - 1, 0, _W - 1)
    return np.stack([r0, r1, c0, c1], axis=1).astype(np.int32)


_RECTS = _rects()


def _erase_body(rect_ref, x_ref, o_ref):
    b = pl.program_id(0)
    r0 = rect_ref[b, 0]
    r1 = rect_ref[b, 1]
    c0 = rect_ref[b, 2]
    c1 = rect_ref[b, 3]
    rows = jax.lax.broadcasted_iota(jnp.int32, (_H, _W), 0)
    cols = jax.lax.broadcasted_iota(jnp.int32, (_H, _W), 1)
    inside = (rows >= r0) & (rows <= r1) & (cols >= c0) & (cols <= c1)
    keep = jnp.where(inside, 0.0, 1.0).astype(x_ref.dtype)
    o_ref[...] = x_ref[...] * keep[None, :, :]


@jax.jit
def kernel(frames):
    n, t, c, h, w = frames.shape
    f = frames.reshape(n * t, c, h, w)
    rects = jnp.asarray(_RECTS)
    out = pl.pallas_call(
        _erase_body,
        grid=(_B,),
        in_specs=[
            pl.BlockSpec(memory_space=pltpu.SMEM),
            pl.BlockSpec((1, c, h, w), lambda b: (b, 0, 0, 0)),
        ],
        out_specs=pl.BlockSpec((1, c, h, w), lambda b: (b, 0, 0, 0)),
        out_shape=jax.ShapeDtypeStruct(f.shape, f.dtype),
    )(rects, f)
    return out.reshape(n, t, c, h, w)


# speedup vs baseline: 99.1864x; 1.7580x over previous
import functools

import jax
import jax.numpy as jnp
import numpy as np
from jax.experimental import pallas as pl
from jax.experimental.pallas import tpu as pltpu

_N, _T, _C, _H, _W = 8, 16, 3, 224, 224
_B = _N * _T
_CX = int(_H * 0.5 + 0.5)
_CY = int(_W * 0.5 + 0.5)


def _rects() -> np.ndarray:
    key = jax.random.key(42)
    kx, ky = jax.random.split(key)
    ox = np.asarray(jax.random.randint(kx, (_B,), 0, _H + (1 - _CX % 2)))
    oy = np.asarray(jax.random.randint(ky, (_B,), 0, _W + (1 - _CY % 2)))
    r0 = np.clip(ox - _CX // 2, 0, _H - 1)
    r1 = np.clip(ox - _CX // 2 + _CX - 1, 0, _H - 1)
    c0 = np.clip(oy - _CY // 2, 0, _W - 1)
    c1 = np.clip(oy - _CY // 2 + _CY - 1, 0, _W - 1)
    return np.stack([r0, r1, c0, c1], axis=1).astype(np.int32)


_RECTS = _rects()


_BLOCK_B = 4


def _erase_body(rect_ref, x_ref, o_ref):
    g = pl.program_id(0)
    rows = jax.lax.broadcasted_iota(jnp.int32, (_H, _W), 0)
    cols = jax.lax.broadcasted_iota(jnp.int32, (_H, _W), 1)
    for i in range(_BLOCK_B):
        b = g * _BLOCK_B + i
        r0 = rect_ref[b, 0]
        r1 = rect_ref[b, 1]
        c0 = rect_ref[b, 2]
        c1 = rect_ref[b, 3]
        inside = (rows >= r0) & (rows <= r1) & (cols >= c0) & (cols <= c1)
        o_ref[i] = jnp.where(inside[None, :, :], 0.0, x_ref[i])


@jax.jit
def kernel(frames):
    n, t, c, h, w = frames.shape
    f = frames.reshape(n * t, c, h, w)
    rects = jnp.asarray(_RECTS)
    out = pl.pallas_call(
        _erase_body,
        grid=(_B // _BLOCK_B,),
        in_specs=[
            pl.BlockSpec(memory_space=pltpu.SMEM),
            pl.BlockSpec((_BLOCK_B, c, h, w), lambda b: (b, 0, 0, 0)),
        ],
        out_specs=pl.BlockSpec((_BLOCK_B, c, h, w), lambda b: (b, 0, 0, 0)),
        out_shape=jax.ShapeDtypeStruct(f.shape, f.dtype),
    )(rects, f)
    return out.reshape(n, t, c, h, w)


# speedup vs baseline: 106.4414x; 1.0731x over previous
import functools

import jax
import jax.numpy as jnp
import numpy as np
from jax.experimental import pallas as pl
from jax.experimental.pallas import tpu as pltpu

_N, _T, _C, _H, _W = 8, 16, 3, 224, 224
_B = _N * _T
_CX = int(_H * 0.5 + 0.5)
_CY = int(_W * 0.5 + 0.5)


def _rects() -> np.ndarray:
    key = jax.random.key(42)
    kx, ky = jax.random.split(key)
    ox = np.asarray(jax.random.randint(kx, (_B,), 0, _H + (1 - _CX % 2)))
    oy = np.asarray(jax.random.randint(ky, (_B,), 0, _W + (1 - _CY % 2)))
    r0 = np.clip(ox - _CX // 2, 0, _H - 1)
    r1 = np.clip(ox - _CX // 2 + _CX - 1, 0, _H - 1)
    c0 = np.clip(oy - _CY // 2, 0, _W - 1)
    c1 = np.clip(oy - _CY // 2 + _CY - 1, 0, _W - 1)
    return np.stack([r0, r1, c0, c1], axis=1).astype(np.int32)


_RECTS = _rects()


_BLOCK_B = 8


def _erase_body(rect_ref, x_ref, o_ref):
    g = pl.program_id(0)
    rows = jax.lax.broadcasted_iota(jnp.int32, (_H, _W), 0)
    cols = jax.lax.broadcasted_iota(jnp.int32, (_H, _W), 1)
    for i in range(_BLOCK_B):
        b = g * _BLOCK_B + i
        r0 = rect_ref[b, 0]
        r1 = rect_ref[b, 1]
        c0 = rect_ref[b, 2]
        c1 = rect_ref[b, 3]
        inside = (rows >= r0) & (rows <= r1) & (cols >= c0) & (cols <= c1)
        o_ref[i] = jnp.where(inside[None, :, :], 0.0, x_ref[i])


@jax.jit
def kernel(frames):
    n, t, c, h, w = frames.shape
    f = frames.reshape(n * t, c, h, w)
    rects = jnp.asarray(_RECTS)
    out = pl.pallas_call(
        _erase_body,
        grid=(_B // _BLOCK_B,),
        in_specs=[
            pl.BlockSpec(memory_space=pltpu.SMEM),
            pl.BlockSpec((_BLOCK_B, c, h, w), lambda b: (b, 0, 0, 0)),
        ],
        out_specs=pl.BlockSpec((_BLOCK_B, c, h, w), lambda b: (b, 0, 0, 0)),
        out_shape=jax.ShapeDtypeStruct(f.shape, f.dtype),
    )(rects, f)
    return out.reshape(n, t, c, h, w)


# speedup vs baseline: 109.0498x; 1.0245x over previous
import functools

import jax
import jax.numpy as jnp
import numpy as np
from jax.experimental import pallas as pl
from jax.experimental.pallas import tpu as pltpu

_N, _T, _C, _H, _W = 8, 16, 3, 224, 224
_B = _N * _T
_CX = int(_H * 0.5 + 0.5)
_CY = int(_W * 0.5 + 0.5)


def _rects() -> np.ndarray:
    key = jax.random.key(42)
    kx, ky = jax.random.split(key)
    ox = np.asarray(jax.random.randint(kx, (_B,), 0, _H + (1 - _CX % 2)))
    oy = np.asarray(jax.random.randint(ky, (_B,), 0, _W + (1 - _CY % 2)))
    r0 = np.clip(ox - _CX // 2, 0, _H - 1)
    r1 = np.clip(ox - _CX // 2 + _CX - 1, 0, _H - 1)
    c0 = np.clip(oy - _CY // 2, 0, _W - 1)
    c1 = np.clip(oy - _CY // 2 + _CY - 1, 0, _W - 1)
    return np.stack([r0, r1, c0, c1], axis=1).astype(np.int32)


_RECTS = _rects()


_BLOCK_B = 16


def _erase_body(rect_ref, x_ref, o_ref):
    g = pl.program_id(0)
    rows = jax.lax.broadcasted_iota(jnp.int32, (_H, _W), 0)
    cols = jax.lax.broadcasted_iota(jnp.int32, (_H, _W), 1)
    for i in range(_BLOCK_B):
        b = g * _BLOCK_B + i
        r0 = rect_ref[b, 0]
        r1 = rect_ref[b, 1]
        c0 = rect_ref[b, 2]
        c1 = rect_ref[b, 3]
        inside = (rows >= r0) & (rows <= r1) & (cols >= c0) & (cols <= c1)
        o_ref[i] = jnp.where(inside[None, :, :], 0.0, x_ref[i])


@jax.jit
def kernel(frames):
    n, t, c, h, w = frames.shape
    f = frames.reshape(n * t, c, h, w)
    rects = jnp.asarray(_RECTS)
    out = pl.pallas_call(
        _erase_body,
        grid=(_B // _BLOCK_B,),
        in_specs=[
            pl.BlockSpec(memory_space=pltpu.SMEM),
            pl.BlockSpec((_BLOCK_B, c, h, w), lambda b: (b, 0, 0, 0)),
        ],
        out_specs=pl.BlockSpec((_BLOCK_B, c, h, w), lambda b: (b, 0, 0, 0)),
        out_shape=jax.ShapeDtypeStruct(f.shape, f.dtype),
    )(rects, f)
    return out.reshape(n, t, c, h, w)
